# baseline (speedup 1.0000x reference)
import dataclasses
import functools

import jax
import jax.numpy as jnp
from jax import lax
from jax.experimental import pallas as pl
from jax.experimental.pallas import tpu as pltpu
from jax.experimental.pallas import tpu_sc as plsc

L, N, E, D = 2, 10000, 160000, 128
NC, NS, LANES = 2, 16, 16
E_PER = E // NS
K = 80
NCHUNK = E_PER // K
NBUF = 2
NIBUF = 2 * NBUF
UNROLL = 4
NGRP = 31
ROWS_PER = 624
TAIL_START = NS * ROWS_PER
TAIL_ROWS = N - TAIL_START

_sc_mesh = plsc.VectorSubcoreMesh(core_axis_name="c", subcore_axis_name="s")

_sc_params = pltpu.CompilerParams()
if "needs_layout_passes" in pltpu.CompilerParams.__dataclass_fields__:
    _sc_params = dataclasses.replace(_sc_params, needs_layout_passes=False)


@functools.partial(
    pl.kernel,
    mesh=_sc_mesh,
    compiler_params=_sc_params,
    out_type=jax.ShapeDtypeStruct((L * N, D), jnp.float32),
    scratch_types=[
        tuple(pltpu.VMEM((3, K), jnp.int32) for _ in range(NIBUF)),
        tuple(pltpu.VMEM((K, D), jnp.float32) for _ in range(NBUF)),
        tuple(pltpu.VMEM((K, D), jnp.float32) for _ in range(NBUF)),
        pltpu.VMEM_SHARED((N, D), jnp.float32),
        pltpu.SemaphoreType.DMA((NIBUF,)),
        pltpu.SemaphoreType.DMA((NBUF,)),
        pltpu.SemaphoreType.DMA((NBUF,)),
    ],
)
def _sc_aggregate(x_hbm, blob_hbm, out_hbm,
                  idxw, grows, srows, acc, isem, gsem, ssem):
    c = lax.axis_index("c")
    s = lax.axis_index("s")

    def issue_blob(cidx, i):
        pltpu.async_copy(blob_hbm.at[c, s, cidx], idxw[i], isem.at[i])

    def wait_blob(i):
        pltpu.make_async_copy(blob_hbm.at[c, s, 0], idxw[i], isem.at[i]).wait()

    def issue_gather(cidx, i, b):
        pltpu.async_copy(x_hbm.at[idxw[i].at[0]], grows[b], gsem.at[b])

    def wait_gather(b):
        pltpu.make_async_copy(x_hbm.at[idxw[0].at[0]], grows[b],
                              gsem.at[b]).wait()

    def issue_scatter(i, b):
        pltpu.async_copy(srows[b], acc.at[idxw[i].at[1]], ssem.at[b], add=True)

    def wait_scatter(b):
        pltpu.make_async_copy(srows[b], acc.at[idxw[0].at[1]],
                              ssem.at[b]).wait()

    def compute(i, b):
        gb, sb = grows[b], srows[b]
        wrow = idxw[i].at[2]

        @plsc.parallel_loop(0, K, unroll=8)
        def _(e):
            wbits = plsc.load_gather(wrow, [jnp.full((LANES,), e, jnp.int32)])
            wb = plsc.bitcast(wbits, jnp.float32)
            for col in range(0, D, LANES):
                sb[e, pl.ds(col, LANES)] = gb[e, pl.ds(col, LANES)] * wb

    for q in range(NBUF):
        issue_blob(q, q)
    for q in range(NBUF):
        wait_blob(q)
        issue_gather(q, q, q)

    z = srows[0]

    @plsc.parallel_loop(0, K, unroll=8)
    def _(r):
        for col in range(0, D, LANES):
            z[r, pl.ds(col, LANES)] = jnp.zeros((LANES,), jnp.float32)

    base_row = s * ROWS_PER
    nfull = ROWS_PER // K
    tail = ROWS_PER - nfull * K

    @pl.loop(0, nfull)
    def _(i):
        pltpu.async_copy(z, acc.at[pl.ds(base_row + i * K, K)], ssem.at[0])

    pltpu.async_copy(z.at[pl.ds(0, tail)],
                     acc.at[pl.ds(base_row + nfull * K, tail)], ssem.at[1])

    @pl.when(s == NS - 1)
    def _():
        pltpu.async_copy(z.at[pl.ds(0, TAIL_ROWS)],
                         acc.at[pl.ds(TAIL_START, TAIL_ROWS)], ssem.at[1])

    @pl.loop(0, nfull)
    def _(i):
        pltpu.make_async_copy(z, acc.at[pl.ds(0, K)], ssem.at[0]).wait()

    pltpu.make_async_copy(z.at[pl.ds(0, tail)],
                          acc.at[pl.ds(0, tail)], ssem.at[1]).wait()

    @pl.when(s == NS - 1)
    def _():
        pltpu.make_async_copy(z.at[pl.ds(0, TAIL_ROWS)],
                              acc.at[pl.ds(0, TAIL_ROWS)], ssem.at[1]).wait()

    plsc.subcore_barrier()

    @pl.loop(0, NGRP)
    def _(g):
        base = g * UNROLL
        for j in range(UNROLL):
            p = base + j
            b = j % NBUF
            i = j % NIBUF

            wait_gather(b)

            @pl.when(p >= NBUF)
            def _():
                wait_scatter(b)

            @pl.when(p + NBUF < NCHUNK)
            def _():
                issue_blob(p + NBUF, (j + NBUF) % NIBUF)

            compute(i, b)
            issue_scatter(i, b)

            @pl.when(p + NBUF < NCHUNK)
            def _():
                wait_blob((j + NBUF) % NIBUF)
                issue_gather(p + NBUF, (j + NBUF) % NIBUF, b)

    for p in (NGRP * UNROLL,):
        b = p % NBUF
        i = p % NIBUF
        wait_gather(b)
        wait_scatter(b)
        compute(i, b)
        issue_scatter(i, b)

    wait_scatter(1)
    wait_scatter(0)

    plsc.subcore_barrier()
    pltpu.sync_copy(acc.at[pl.ds(s * ROWS_PER, ROWS_PER)],
                    out_hbm.at[pl.ds(c * N + s * ROWS_PER, ROWS_PER)])

    @pl.when(s == NS - 1)
    def _():
        pltpu.sync_copy(acc.at[pl.ds(TAIL_START, TAIL_ROWS)],
                        out_hbm.at[pl.ds(c * N + TAIL_START, TAIL_ROWS)])


def _tc_body(agg_ref, w_ref, b_ref, g_ref, bt_ref, o_ref):
    h = lax.dot_general(agg_ref[...], w_ref[...],
                        dimension_numbers=(((1,), (1,)), ((), ())),
                        preferred_element_type=jnp.float32)
    h = h + b_ref[...]
    h = h * jax.nn.sigmoid(h)
    mean = jnp.mean(h, axis=1, keepdims=True)
    var = jnp.mean((h - mean) ** 2, axis=1, keepdims=True)
    o_ref[...] = (h - mean) * lax.rsqrt(var + 1e-5) * g_ref[...] + bt_ref[...]


def _tc_postprocess(agg, W, b, gamma, beta):
    BM = 2000
    return pl.pallas_call(
        _tc_body,
        out_shape=jax.ShapeDtypeStruct((L * N, D), jnp.float32),
        grid=((L * N) // BM,),
        in_specs=[
            pl.BlockSpec((BM, D), lambda i: (i, 0)),
            pl.BlockSpec((D, D), lambda i: (0, 0)),
            pl.BlockSpec((1, D), lambda i: (0, 0)),
            pl.BlockSpec((1, D), lambda i: (0, 0)),
            pl.BlockSpec((1, D), lambda i: (0, 0)),
        ],
        out_specs=pl.BlockSpec((BM, D), lambda i: (i, 0)),
    )(agg, W, b.reshape(1, D), gamma.reshape(1, D), beta.reshape(1, D))


def kernel(x, edge_index, edge_weight, W, b, gamma, beta):
    x_flat = x.reshape(L * N, D)
    dst = edge_index[0].reshape(NS, NCHUNK, 1, K)
    src = edge_index[1].reshape(NS, NCHUNK, 1, K)
    w_bits = lax.bitcast_convert_type(edge_weight,
                                      jnp.int32).reshape(NS, NCHUNK, 1, K)
    blob = jnp.stack([
        jnp.concatenate([src, dst, w_bits], axis=2),
        jnp.concatenate([src + N, dst, w_bits], axis=2),
    ])
    agg = _sc_aggregate(x_flat, blob)
    out = _tc_postprocess(agg, W, b, gamma, beta)
    return out.reshape(L, N, D)


# speedup vs baseline: 1.0952x; 1.0952x over previous
import dataclasses
import functools

import jax
import jax.numpy as jnp
from jax import lax
from jax.experimental import pallas as pl
from jax.experimental.pallas import tpu as pltpu
from jax.experimental.pallas import tpu_sc as plsc

L, N, E, D = 2, 10000, 160000, 128
NC, NS, LANES = 2, 16, 16
E_PER = E // NS
K = 80
NCHUNK = E_PER // K
NBUF = 2
NIBUF = 2 * NBUF
UNROLL = 4
NGRP = 31
ROWS_PER = 624
TAIL_START = NS * ROWS_PER
TAIL_ROWS = N - TAIL_START

_sc_mesh = plsc.VectorSubcoreMesh(core_axis_name="c", subcore_axis_name="s")

_sc_params = pltpu.CompilerParams()
if "needs_layout_passes" in pltpu.CompilerParams.__dataclass_fields__:
    _sc_params = dataclasses.replace(_sc_params, needs_layout_passes=False)


@functools.partial(
    pl.kernel,
    mesh=_sc_mesh,
    compiler_params=_sc_params,
    out_type=jax.ShapeDtypeStruct((L * N, D), jnp.float32),
    scratch_types=[
        tuple(pltpu.VMEM((K,), jnp.int32) for _ in range(NIBUF)),
        tuple(pltpu.VMEM((K,), jnp.int32) for _ in range(NIBUF)),
        tuple(pltpu.VMEM((K,), jnp.float32) for _ in range(NIBUF)),
        tuple(pltpu.VMEM((K, D), jnp.float32) for _ in range(NBUF)),
        tuple(pltpu.VMEM((K, D), jnp.float32) for _ in range(NBUF)),
        pltpu.VMEM_SHARED((N, D), jnp.float32),
        pltpu.SemaphoreType.DMA((NIBUF,)),
        pltpu.SemaphoreType.DMA((NBUF,)),
        pltpu.SemaphoreType.DMA((NBUF,)),
    ],
)
def _sc_aggregate(x_hbm, ei_hbm, ew_hbm, out_hbm,
                  srcb, dstb, wvb, grows, srows, acc, isem, gsem, ssem):
    c = lax.axis_index("c")
    s = lax.axis_index("s")
    ebase = s * E_PER
    cN = jnp.full((LANES,), c * N, jnp.int32)

    def issue_blob(cidx, i):
        off = ebase + cidx * K
        pltpu.async_copy(ei_hbm.at[pl.ds(E + off, K)], srcb[i], isem.at[i])
        pltpu.async_copy(ei_hbm.at[pl.ds(off, K)], dstb[i], isem.at[i])
        pltpu.async_copy(ew_hbm.at[pl.ds(off, K)], wvb[i], isem.at[i])

    def wait_blob(i):
        pltpu.make_async_copy(ei_hbm.at[pl.ds(0, K)], srcb[i],
                              isem.at[i]).wait()
        pltpu.make_async_copy(ei_hbm.at[pl.ds(0, K)], dstb[i],
                              isem.at[i]).wait()
        pltpu.make_async_copy(ew_hbm.at[pl.ds(0, K)], wvb[i],
                              isem.at[i]).wait()
        for t in range(K // LANES):
            sl = pl.ds(t * LANES, LANES)
            srcb[i][sl] = srcb[i][sl] + cN

    def issue_gather(cidx, i, b):
        pltpu.async_copy(x_hbm.at[srcb[i]], grows[b], gsem.at[b])

    def wait_gather(b):
        pltpu.make_async_copy(x_hbm.at[srcb[0]], grows[b],
                              gsem.at[b]).wait()

    def issue_scatter(i, b):
        pltpu.async_copy(srows[b], acc.at[dstb[i]], ssem.at[b], add=True)

    def wait_scatter(b):
        pltpu.make_async_copy(srows[b], acc.at[dstb[0]],
                              ssem.at[b]).wait()

    def compute(i, b):
        gb, sb = grows[b], srows[b]
        wrow = wvb[i]

        @plsc.parallel_loop(0, K, unroll=8)
        def _(e):
            wb = plsc.load_gather(wrow, [jnp.full((LANES,), e, jnp.int32)])
            for col in range(0, D, LANES):
                sb[e, pl.ds(col, LANES)] = gb[e, pl.ds(col, LANES)] * wb

    for q in range(NBUF):
        issue_blob(q, q)
    for q in range(NBUF):
        wait_blob(q)
        issue_gather(q, q, q)

    z = srows[0]

    @plsc.parallel_loop(0, K, unroll=8)
    def _(r):
        for col in range(0, D, LANES):
            z[r, pl.ds(col, LANES)] = jnp.zeros((LANES,), jnp.float32)

    base_row = s * ROWS_PER
    nfull = ROWS_PER // K
    tail = ROWS_PER - nfull * K

    @pl.loop(0, nfull)
    def _(i):
        pltpu.async_copy(z, acc.at[pl.ds(base_row + i * K, K)], ssem.at[0])

    pltpu.async_copy(z.at[pl.ds(0, tail)],
                     acc.at[pl.ds(base_row + nfull * K, tail)], ssem.at[1])

    @pl.when(s == NS - 1)
    def _():
        pltpu.async_copy(z.at[pl.ds(0, TAIL_ROWS)],
                         acc.at[pl.ds(TAIL_START, TAIL_ROWS)], ssem.at[1])

    @pl.loop(0, nfull)
    def _(i):
        pltpu.make_async_copy(z, acc.at[pl.ds(0, K)], ssem.at[0]).wait()

    pltpu.make_async_copy(z.at[pl.ds(0, tail)],
                          acc.at[pl.ds(0, tail)], ssem.at[1]).wait()

    @pl.when(s == NS - 1)
    def _():
        pltpu.make_async_copy(z.at[pl.ds(0, TAIL_ROWS)],
                              acc.at[pl.ds(0, TAIL_ROWS)], ssem.at[1]).wait()

    plsc.subcore_barrier()

    @pl.loop(0, NGRP)
    def _(g):
        base = g * UNROLL
        for j in range(UNROLL):
            p = base + j
            b = j % NBUF
            i = j % NIBUF

            wait_gather(b)

            @pl.when(p >= NBUF)
            def _():
                wait_scatter(b)

            @pl.when(p + NBUF < NCHUNK)
            def _():
                issue_blob(p + NBUF, (j + NBUF) % NIBUF)

            compute(i, b)
            issue_scatter(i, b)

            @pl.when(p + NBUF < NCHUNK)
            def _():
                wait_blob((j + NBUF) % NIBUF)
                issue_gather(p + NBUF, (j + NBUF) % NIBUF, b)

    for p in (NGRP * UNROLL,):
        b = p % NBUF
        i = p % NIBUF
        wait_gather(b)
        wait_scatter(b)
        compute(i, b)
        issue_scatter(i, b)

    wait_scatter(1)
    wait_scatter(0)

    plsc.subcore_barrier()
    pltpu.sync_copy(acc.at[pl.ds(s * ROWS_PER, ROWS_PER)],
                    out_hbm.at[pl.ds(c * N + s * ROWS_PER, ROWS_PER)])

    @pl.when(s == NS - 1)
    def _():
        pltpu.sync_copy(acc.at[pl.ds(TAIL_START, TAIL_ROWS)],
                        out_hbm.at[pl.ds(c * N + TAIL_START, TAIL_ROWS)])


def _tc_body(agg_ref, w_ref, b_ref, g_ref, bt_ref, o_ref):
    h = lax.dot_general(agg_ref[...], w_ref[...],
                        dimension_numbers=(((1,), (1,)), ((), ())),
                        preferred_element_type=jnp.float32)
    h = h + b_ref[...]
    h = h * jax.nn.sigmoid(h)
    mean = jnp.mean(h, axis=1, keepdims=True)
    var = jnp.mean((h - mean) ** 2, axis=1, keepdims=True)
    o_ref[...] = (h - mean) * lax.rsqrt(var + 1e-5) * g_ref[...] + bt_ref[...]


def _tc_postprocess(agg, W, b, gamma, beta):
    BM = 2000
    return pl.pallas_call(
        _tc_body,
        out_shape=jax.ShapeDtypeStruct((L * N, D), jnp.float32),
        grid=((L * N) // BM,),
        in_specs=[
            pl.BlockSpec((BM, D), lambda i: (i, 0)),
            pl.BlockSpec((D, D), lambda i: (0, 0)),
            pl.BlockSpec((1, D), lambda i: (0, 0)),
            pl.BlockSpec((1, D), lambda i: (0, 0)),
            pl.BlockSpec((1, D), lambda i: (0, 0)),
        ],
        out_specs=pl.BlockSpec((BM, D), lambda i: (i, 0)),
    )(agg, W, b.reshape(1, D), gamma.reshape(1, D), beta.reshape(1, D))


def kernel(x, edge_index, edge_weight, W, b, gamma, beta):
    x_flat = x.reshape(L * N, D)
    agg = _sc_aggregate(x_flat, edge_index.reshape(2 * E), edge_weight)
    out = _tc_postprocess(agg, W, b, gamma, beta)
    return out.reshape(L, N, D)


# speedup vs baseline: 1.1138x; 1.0169x over previous
import dataclasses
import functools

import jax
import jax.numpy as jnp
from jax import lax
from jax.experimental import pallas as pl
from jax.experimental.pallas import tpu as pltpu
from jax.experimental.pallas import tpu_sc as plsc

L, N, E, D = 2, 10000, 160000, 128
NC, NS, LANES = 2, 16, 16
E_PER = E // NS
K = 80
NCHUNK = E_PER // K
NBUF = 2
NIBUF = 2 * NBUF
UNROLL = 4
NGRP = 31
ROWS_PER = 624
TAIL_START = NS * ROWS_PER
TAIL_ROWS = N - TAIL_START

_sc_mesh = plsc.VectorSubcoreMesh(core_axis_name="c", subcore_axis_name="s")

_sc_params = pltpu.CompilerParams()
if "needs_layout_passes" in pltpu.CompilerParams.__dataclass_fields__:
    _sc_params = dataclasses.replace(_sc_params, needs_layout_passes=False)


@functools.partial(
    pl.kernel,
    mesh=_sc_mesh,
    compiler_params=_sc_params,
    out_type=jax.ShapeDtypeStruct((L * N, D), jnp.float32),
    scratch_types=[
        tuple(pltpu.VMEM((K,), jnp.int32) for _ in range(NIBUF)),
        tuple(pltpu.VMEM((K,), jnp.int32) for _ in range(NIBUF)),
        tuple(pltpu.VMEM((K,), jnp.float32) for _ in range(NIBUF)),
        tuple(pltpu.VMEM((K, D), jnp.float32) for _ in range(NBUF)),
        tuple(pltpu.VMEM((K, D), jnp.float32) for _ in range(NBUF)),
        pltpu.VMEM_SHARED((N, D), jnp.float32),
        pltpu.SemaphoreType.DMA((NIBUF,)),
        pltpu.SemaphoreType.DMA((NBUF,)),
        pltpu.SemaphoreType.DMA((NBUF,)),
    ],
)
def _sc_aggregate(x_hbm, ei_hbm, ew_hbm, out_hbm,
                  srcb, dstb, wvb, grows, srows, acc, isem, gsem, ssem):
    c = lax.axis_index("c")
    s = lax.axis_index("s")
    ebase = s * E_PER
    cN = jnp.full((LANES,), c * N, jnp.int32)

    def issue_blob(cidx, i):
        off = ebase + cidx * K
        pltpu.async_copy(ei_hbm.at[pl.ds(E + off, K)], srcb[i], isem.at[i])
        pltpu.async_copy(ei_hbm.at[pl.ds(off, K)], dstb[i], isem.at[i])
        pltpu.async_copy(ew_hbm.at[pl.ds(off, K)], wvb[i], isem.at[i])

    def wait_blob(i):
        pltpu.make_async_copy(ei_hbm.at[pl.ds(0, K)], srcb[i],
                              isem.at[i]).wait()
        pltpu.make_async_copy(ei_hbm.at[pl.ds(0, K)], dstb[i],
                              isem.at[i]).wait()
        pltpu.make_async_copy(ew_hbm.at[pl.ds(0, K)], wvb[i],
                              isem.at[i]).wait()
        for t in range(K // LANES):
            sl = pl.ds(t * LANES, LANES)
            srcb[i][sl] = srcb[i][sl] + cN

    def issue_gather(cidx, i, b):
        pltpu.async_copy(x_hbm.at[srcb[i]], grows[b], gsem.at[b])

    def wait_gather(b):
        pltpu.make_async_copy(x_hbm.at[srcb[0]], grows[b],
                              gsem.at[b]).wait()

    def issue_scatter(i, b):
        pltpu.async_copy(srows[b], acc.at[dstb[i]], ssem.at[b], add=True)

    def wait_scatter(b):
        pltpu.make_async_copy(srows[b], acc.at[dstb[0]],
                              ssem.at[b]).wait()

    def compute(i, b):
        gb, sb = grows[b], srows[b]
        wrow = wvb[i]

        @plsc.parallel_loop(0, K, unroll=8)
        def _(e):
            wb = plsc.load_gather(wrow, [jnp.full((LANES,), e, jnp.int32)])
            for col in range(0, D, LANES):
                sb[e, pl.ds(col, LANES)] = gb[e, pl.ds(col, LANES)] * wb

    for q in range(NBUF):
        issue_blob(q, q)
    for q in range(NBUF):
        wait_blob(q)
        issue_gather(q, q, q)

    z = srows[0]

    @plsc.parallel_loop(0, K, unroll=8)
    def _(r):
        for col in range(0, D, LANES):
            z[r, pl.ds(col, LANES)] = jnp.zeros((LANES,), jnp.float32)

    base_row = s * ROWS_PER
    nfull = ROWS_PER // K
    tail = ROWS_PER - nfull * K

    @pl.loop(0, nfull)
    def _(i):
        pltpu.async_copy(z, acc.at[pl.ds(base_row + i * K, K)], ssem.at[0])

    pltpu.async_copy(z.at[pl.ds(0, tail)],
                     acc.at[pl.ds(base_row + nfull * K, tail)], ssem.at[1])

    @pl.when(s == NS - 1)
    def _():
        pltpu.async_copy(z.at[pl.ds(0, TAIL_ROWS)],
                         acc.at[pl.ds(TAIL_START, TAIL_ROWS)], ssem.at[1])

    @pl.loop(0, nfull)
    def _(i):
        pltpu.make_async_copy(z, acc.at[pl.ds(0, K)], ssem.at[0]).wait()

    pltpu.make_async_copy(z.at[pl.ds(0, tail)],
                          acc.at[pl.ds(0, tail)], ssem.at[1]).wait()

    @pl.when(s == NS - 1)
    def _():
        pltpu.make_async_copy(z.at[pl.ds(0, TAIL_ROWS)],
                              acc.at[pl.ds(0, TAIL_ROWS)], ssem.at[1]).wait()

    plsc.subcore_barrier()

    @pl.loop(0, NGRP)
    def _(g):
        base = g * UNROLL
        for j in range(UNROLL):
            p = base + j
            b = j % NBUF
            i = j % NIBUF

            wait_gather(b)

            @pl.when(p >= NBUF)
            def _():
                wait_scatter(b)

            @pl.when(p + NBUF < NCHUNK)
            def _():
                issue_blob(p + NBUF, (j + NBUF) % NIBUF)

            compute(i, b)
            issue_scatter(i, b)

            @pl.when(p + NBUF < NCHUNK)
            def _():
                wait_blob((j + NBUF) % NIBUF)
                issue_gather(p + NBUF, (j + NBUF) % NIBUF, b)

    for p in (NGRP * UNROLL,):
        b = p % NBUF
        i = p % NIBUF
        wait_gather(b)
        wait_scatter(b)
        compute(i, b)
        issue_scatter(i, b)

    wait_scatter(1)
    wait_scatter(0)

    plsc.subcore_barrier()
    pltpu.sync_copy(acc.at[pl.ds(s * ROWS_PER, ROWS_PER)],
                    out_hbm.at[pl.ds(c * N + s * ROWS_PER, ROWS_PER)])

    @pl.when(s == NS - 1)
    def _():
        pltpu.sync_copy(acc.at[pl.ds(TAIL_START, TAIL_ROWS)],
                        out_hbm.at[pl.ds(c * N + TAIL_START, TAIL_ROWS)])


def _tc_body(agg_ref, w_ref, b_ref, g_ref, bt_ref, o_ref):
    h = lax.dot_general(agg_ref[...], w_ref[...],
                        dimension_numbers=(((1,), (1,)), ((), ())),
                        preferred_element_type=jnp.float32)
    h = h + b_ref[...]
    h = h * jax.nn.sigmoid(h)
    mean = jnp.mean(h, axis=1, keepdims=True)
    var = jnp.mean((h - mean) ** 2, axis=1, keepdims=True)
    o_ref[...] = (h - mean) * lax.rsqrt(var + 1e-5) * g_ref[...] + bt_ref[...]


def _tc_postprocess(agg, W, b, gamma, beta):
    BM = 5000
    return pl.pallas_call(
        _tc_body,
        out_shape=jax.ShapeDtypeStruct((L * N, D), jnp.float32),
        grid=((L * N) // BM,),
        in_specs=[
            pl.BlockSpec((BM, D), lambda i: (i, 0)),
            pl.BlockSpec((D, D), lambda i: (0, 0)),
            pl.BlockSpec((1, D), lambda i: (0, 0)),
            pl.BlockSpec((1, D), lambda i: (0, 0)),
            pl.BlockSpec((1, D), lambda i: (0, 0)),
        ],
        out_specs=pl.BlockSpec((BM, D), lambda i: (i, 0)),
    )(agg, W, b.reshape(1, D), gamma.reshape(1, D), beta.reshape(1, D))


def kernel(x, edge_index, edge_weight, W, b, gamma, beta):
    x_flat = x.reshape(L * N, D)
    agg = _sc_aggregate(x_flat, edge_index.reshape(2 * E), edge_weight)
    out = _tc_postprocess(agg, W, b, gamma, beta)
    return out.reshape(L, N, D)


# speedup vs baseline: 1.1144x; 1.0005x over previous
import dataclasses
import functools

import jax
import jax.numpy as jnp
from jax import lax
from jax.experimental import pallas as pl
from jax.experimental.pallas import tpu as pltpu
from jax.experimental.pallas import tpu_sc as plsc

L, N, E, D = 2, 10000, 160000, 128
NC, NS, LANES = 2, 16, 16
E_PER = E // NS
K = 80
NCHUNK = E_PER // K
NBUF = 2
NIBUF = 2 * NBUF
UNROLL = 4
NGRP = 31
ROWS_PER = 624
TAIL_START = NS * ROWS_PER
TAIL_ROWS = N - TAIL_START

_sc_mesh = plsc.VectorSubcoreMesh(core_axis_name="c", subcore_axis_name="s")

_sc_params = pltpu.CompilerParams()
if "needs_layout_passes" in pltpu.CompilerParams.__dataclass_fields__:
    _sc_params = dataclasses.replace(_sc_params, needs_layout_passes=False)


@functools.partial(
    pl.kernel,
    mesh=_sc_mesh,
    compiler_params=_sc_params,
    out_type=jax.ShapeDtypeStruct((L * N, D), jnp.float32),
    scratch_types=[
        tuple(pltpu.VMEM((K,), jnp.int32) for _ in range(NIBUF)),
        tuple(pltpu.VMEM((K,), jnp.int32) for _ in range(NIBUF)),
        tuple(pltpu.VMEM((K,), jnp.float32) for _ in range(NIBUF)),
        tuple(pltpu.VMEM((K, D), jnp.float32) for _ in range(NBUF)),
        tuple(pltpu.VMEM((K, D), jnp.float32) for _ in range(NBUF)),
        pltpu.VMEM_SHARED((N, D), jnp.float32),
        pltpu.SemaphoreType.DMA((NIBUF,)),
        pltpu.SemaphoreType.DMA((NBUF,)),
        pltpu.SemaphoreType.DMA((NBUF,)),
    ],
)
def _sc_aggregate(x_hbm, ei_hbm, ew_hbm, out_hbm,
                  srcb, dstb, wvb, grows, srows, acc, isem, gsem, ssem):
    c = lax.axis_index("c")
    s = lax.axis_index("s")
    ebase = s * E_PER
    cN = jnp.full((LANES,), c * N, jnp.int32)

    def issue_blob(cidx, i):
        off = ebase + cidx * K
        pltpu.async_copy(ei_hbm.at[pl.ds(E + off, K)], srcb[i], isem.at[i])
        pltpu.async_copy(ei_hbm.at[pl.ds(off, K)], dstb[i], isem.at[i])
        pltpu.async_copy(ew_hbm.at[pl.ds(off, K)], wvb[i], isem.at[i])

    def wait_blob(i):
        pltpu.make_async_copy(ei_hbm.at[pl.ds(0, K)], srcb[i],
                              isem.at[i]).wait()
        pltpu.make_async_copy(ei_hbm.at[pl.ds(0, K)], dstb[i],
                              isem.at[i]).wait()
        pltpu.make_async_copy(ew_hbm.at[pl.ds(0, K)], wvb[i],
                              isem.at[i]).wait()
        for t in range(K // LANES):
            sl = pl.ds(t * LANES, LANES)
            srcb[i][sl] = srcb[i][sl] + cN

    def issue_gather(cidx, i, b):
        pltpu.async_copy(x_hbm.at[srcb[i]], grows[b], gsem.at[b])

    def wait_gather(b):
        pltpu.make_async_copy(x_hbm.at[srcb[0]], grows[b],
                              gsem.at[b]).wait()

    def issue_scatter(i, b):
        pltpu.async_copy(srows[b], acc.at[dstb[i]], ssem.at[b], add=True)

    def wait_scatter(b):
        pltpu.make_async_copy(srows[b], acc.at[dstb[0]],
                              ssem.at[b]).wait()

    def compute(i, b):
        gb, sb = grows[b], srows[b]
        wrow = wvb[i]

        @plsc.parallel_loop(0, K, unroll=8)
        def _(e):
            wb = plsc.load_gather(wrow, [jnp.full((LANES,), e, jnp.int32)])
            for col in range(0, D, LANES):
                sb[e, pl.ds(col, LANES)] = gb[e, pl.ds(col, LANES)] * wb

    for q in range(NBUF):
        issue_blob(q, q)
    for q in range(NBUF):
        wait_blob(q)
        issue_gather(q, q, q)

    z = srows[0]

    @plsc.parallel_loop(0, K, unroll=8)
    def _(r):
        for col in range(0, D, LANES):
            z[r, pl.ds(col, LANES)] = jnp.zeros((LANES,), jnp.float32)

    base_row = s * ROWS_PER
    nfull = ROWS_PER // K
    tail = ROWS_PER - nfull * K

    @pl.loop(0, nfull)
    def _(i):
        pltpu.async_copy(z, acc.at[pl.ds(base_row + i * K, K)], ssem.at[0])

    pltpu.async_copy(z.at[pl.ds(0, tail)],
                     acc.at[pl.ds(base_row + nfull * K, tail)], ssem.at[1])

    @pl.when(s == NS - 1)
    def _():
        pltpu.async_copy(z.at[pl.ds(0, TAIL_ROWS)],
                         acc.at[pl.ds(TAIL_START, TAIL_ROWS)], ssem.at[1])

    @pl.loop(0, nfull)
    def _(i):
        pltpu.make_async_copy(z, acc.at[pl.ds(0, K)], ssem.at[0]).wait()

    pltpu.make_async_copy(z.at[pl.ds(0, tail)],
                          acc.at[pl.ds(0, tail)], ssem.at[1]).wait()

    @pl.when(s == NS - 1)
    def _():
        pltpu.make_async_copy(z.at[pl.ds(0, TAIL_ROWS)],
                              acc.at[pl.ds(0, TAIL_ROWS)], ssem.at[1]).wait()

    plsc.subcore_barrier()

    @pl.loop(0, NGRP)
    def _(g):
        base = g * UNROLL
        for j in range(UNROLL):
            p = base + j
            b = j % NBUF
            i = j % NIBUF

            wait_gather(b)

            @pl.when(p >= NBUF)
            def _():
                wait_scatter(b)

            @pl.when(p + NBUF < NCHUNK)
            def _():
                issue_blob(p + NBUF, (j + NBUF) % NIBUF)

            compute(i, b)
            issue_scatter(i, b)

            @pl.when(p + NBUF < NCHUNK)
            def _():
                wait_blob((j + NBUF) % NIBUF)
                issue_gather(p + NBUF, (j + NBUF) % NIBUF, b)

    for p in (NGRP * UNROLL,):
        b = p % NBUF
        i = p % NIBUF
        wait_gather(b)
        wait_scatter(b)
        compute(i, b)
        issue_scatter(i, b)

    wait_scatter(1)
    wait_scatter(0)

    plsc.subcore_barrier()
    pltpu.sync_copy(acc.at[pl.ds(s * ROWS_PER, ROWS_PER)],
                    out_hbm.at[pl.ds(c * N + s * ROWS_PER, ROWS_PER)])

    @pl.when(s == NS - 1)
    def _():
        pltpu.sync_copy(acc.at[pl.ds(TAIL_START, TAIL_ROWS)],
                        out_hbm.at[pl.ds(c * N + TAIL_START, TAIL_ROWS)])


def _tc_body(agg_ref, w_ref, b_ref, g_ref, bt_ref, o_ref):
    h = lax.dot_general(agg_ref[...], w_ref[...],
                        dimension_numbers=(((1,), (1,)), ((), ())),
                        preferred_element_type=jnp.float32)
    h = h + b_ref[...]
    h = h * jax.nn.sigmoid(h)
    mean = jnp.mean(h, axis=1, keepdims=True)
    var = jnp.mean((h - mean) ** 2, axis=1, keepdims=True)
    o_ref[...] = (h - mean) * lax.rsqrt(var + 1e-5) * g_ref[...] + bt_ref[...]


def _tc_postprocess(agg, W, b, gamma, beta):
    BM = 10000
    return pl.pallas_call(
        _tc_body,
        out_shape=jax.ShapeDtypeStruct((L * N, D), jnp.float32),
        grid=((L * N) // BM,),
        in_specs=[
            pl.BlockSpec((BM, D), lambda i: (i, 0)),
            pl.BlockSpec((D, D), lambda i: (0, 0)),
            pl.BlockSpec((1, D), lambda i: (0, 0)),
            pl.BlockSpec((1, D), lambda i: (0, 0)),
            pl.BlockSpec((1, D), lambda i: (0, 0)),
        ],
        out_specs=pl.BlockSpec((BM, D), lambda i: (i, 0)),
    )(agg, W, b.reshape(1, D), gamma.reshape(1, D), beta.reshape(1, D))


def kernel(x, edge_index, edge_weight, W, b, gamma, beta):
    x_flat = x.reshape(L * N, D)
    agg = _sc_aggregate(x_flat, edge_index.reshape(2 * E), edge_weight)
    out = _tc_postprocess(agg, W, b, gamma, beta)
    return out.reshape(L, N, D)


# speedup vs baseline: 1.1379x; 1.0212x over previous
import dataclasses
import functools

import jax
import jax.numpy as jnp
from jax import lax
from jax.experimental import pallas as pl
from jax.experimental.pallas import tpu as pltpu
from jax.experimental.pallas import tpu_sc as plsc

L, N, E, D = 2, 10000, 160000, 128
NC, NS, LANES = 2, 16, 16
E_PER = E // NS
K = 80
NCHUNK = E_PER // K
NBUF = 2
NIBUF = 8
UNROLL = 8
NGRP = 15
ROWS_PER = 624
TAIL_START = NS * ROWS_PER
TAIL_ROWS = N - TAIL_START

_sc_mesh = plsc.VectorSubcoreMesh(core_axis_name="c", subcore_axis_name="s")

_sc_params = pltpu.CompilerParams()
if "needs_layout_passes" in pltpu.CompilerParams.__dataclass_fields__:
    _sc_params = dataclasses.replace(_sc_params, needs_layout_passes=False)


@functools.partial(
    pl.kernel,
    mesh=_sc_mesh,
    compiler_params=_sc_params,
    out_type=jax.ShapeDtypeStruct((L * N, D), jnp.float32),
    scratch_types=[
        tuple(pltpu.VMEM((K,), jnp.int32) for _ in range(NIBUF)),
        tuple(pltpu.VMEM((K,), jnp.int32) for _ in range(NIBUF)),
        tuple(pltpu.VMEM((K,), jnp.float32) for _ in range(NIBUF)),
        tuple(pltpu.VMEM((K, D), jnp.float32) for _ in range(NBUF)),
        tuple(pltpu.VMEM((K, D), jnp.float32) for _ in range(NBUF)),
        pltpu.VMEM_SHARED((N, D), jnp.float32),
        pltpu.SemaphoreType.DMA((NIBUF,)),
        pltpu.SemaphoreType.DMA((NBUF,)),
        pltpu.SemaphoreType.DMA((NBUF,)),
    ],
)
def _sc_aggregate(x_hbm, ei_hbm, ew_hbm, out_hbm,
                  srcb, dstb, wvb, grows, srows, acc, isem, gsem, ssem):
    c = lax.axis_index("c")
    s = lax.axis_index("s")
    ebase = s * E_PER
    cN = jnp.full((LANES,), c * N, jnp.int32)

    def issue_blob(cidx, i):
        off = ebase + cidx * K
        pltpu.async_copy(ei_hbm.at[pl.ds(E + off, K)], srcb[i], isem.at[i])
        pltpu.async_copy(ei_hbm.at[pl.ds(off, K)], dstb[i], isem.at[i])
        pltpu.async_copy(ew_hbm.at[pl.ds(off, K)], wvb[i], isem.at[i])

    def wait_blob(i):
        pltpu.make_async_copy(ei_hbm.at[pl.ds(0, K)], srcb[i],
                              isem.at[i]).wait()
        pltpu.make_async_copy(ei_hbm.at[pl.ds(0, K)], dstb[i],
                              isem.at[i]).wait()
        pltpu.make_async_copy(ew_hbm.at[pl.ds(0, K)], wvb[i],
                              isem.at[i]).wait()
        for t in range(K // LANES):
            sl = pl.ds(t * LANES, LANES)
            srcb[i][sl] = srcb[i][sl] + cN

    def issue_gather(cidx, i, b):
        pltpu.async_copy(x_hbm.at[srcb[i]], grows[b], gsem.at[b])

    def wait_gather(b):
        pltpu.make_async_copy(x_hbm.at[srcb[0]], grows[b],
                              gsem.at[b]).wait()

    def issue_scatter(i, b):
        pltpu.async_copy(srows[b], acc.at[dstb[i]], ssem.at[b], add=True)

    def wait_scatter(b):
        pltpu.make_async_copy(srows[b], acc.at[dstb[0]],
                              ssem.at[b]).wait()

    def compute(i, b):
        gb, sb = grows[b], srows[b]
        wrow = wvb[i]

        @plsc.parallel_loop(0, K, unroll=8)
        def _(e):
            wb = plsc.load_gather(wrow, [jnp.full((LANES,), e, jnp.int32)])
            for col in range(0, D, LANES):
                sb[e, pl.ds(col, LANES)] = gb[e, pl.ds(col, LANES)] * wb

    for q in range(6):
        issue_blob(q, q)
    for q in range(NBUF):
        wait_blob(q)
        issue_gather(q, q, q)

    z = srows[0]

    @plsc.parallel_loop(0, K, unroll=8)
    def _(r):
        for col in range(0, D, LANES):
            z[r, pl.ds(col, LANES)] = jnp.zeros((LANES,), jnp.float32)

    base_row = s * ROWS_PER
    nfull = ROWS_PER // K
    tail = ROWS_PER - nfull * K

    @pl.loop(0, nfull)
    def _(i):
        pltpu.async_copy(z, acc.at[pl.ds(base_row + i * K, K)], ssem.at[0])

    pltpu.async_copy(z.at[pl.ds(0, tail)],
                     acc.at[pl.ds(base_row + nfull * K, tail)], ssem.at[1])

    @pl.when(s == NS - 1)
    def _():
        pltpu.async_copy(z.at[pl.ds(0, TAIL_ROWS)],
                         acc.at[pl.ds(TAIL_START, TAIL_ROWS)], ssem.at[1])

    @pl.loop(0, nfull)
    def _(i):
        pltpu.make_async_copy(z, acc.at[pl.ds(0, K)], ssem.at[0]).wait()

    pltpu.make_async_copy(z.at[pl.ds(0, tail)],
                          acc.at[pl.ds(0, tail)], ssem.at[1]).wait()

    @pl.when(s == NS - 1)
    def _():
        pltpu.make_async_copy(z.at[pl.ds(0, TAIL_ROWS)],
                              acc.at[pl.ds(0, TAIL_ROWS)], ssem.at[1]).wait()

    plsc.subcore_barrier()

    @pl.loop(0, NGRP)
    def _(g):
        base = g * UNROLL
        for j in range(UNROLL):
            p = base + j
            b = j % NBUF
            i = j % NIBUF

            wait_gather(b)

            @pl.when(p >= NBUF)
            def _():
                wait_scatter(b)

            @pl.when(p + 6 < NCHUNK)
            def _():
                issue_blob(p + 6, (j + 6) % NIBUF)

            compute(i, b)
            issue_scatter(i, b)

            @pl.when(p + NBUF < NCHUNK)
            def _():
                wait_blob((j + NBUF) % NIBUF)
                issue_gather(p + NBUF, (j + NBUF) % NIBUF, b)

    for p in range(NGRP * UNROLL, NCHUNK):
        j = p % UNROLL
        b = j % NBUF
        i = j % NIBUF
        wait_gather(b)
        wait_scatter(b)
        compute(i, b)
        issue_scatter(i, b)
        if p + NBUF < NCHUNK:
            wait_blob((j + NBUF) % NIBUF)
            issue_gather(p + NBUF, (j + NBUF) % NIBUF, b)

    wait_scatter(1)
    wait_scatter(0)

    plsc.subcore_barrier()
    pltpu.sync_copy(acc.at[pl.ds(s * ROWS_PER, ROWS_PER)],
                    out_hbm.at[pl.ds(c * N + s * ROWS_PER, ROWS_PER)])

    @pl.when(s == NS - 1)
    def _():
        pltpu.sync_copy(acc.at[pl.ds(TAIL_START, TAIL_ROWS)],
                        out_hbm.at[pl.ds(c * N + TAIL_START, TAIL_ROWS)])


def _tc_body(agg_ref, w_ref, b_ref, g_ref, bt_ref, o_ref):
    h = lax.dot_general(agg_ref[...], w_ref[...],
                        dimension_numbers=(((1,), (1,)), ((), ())),
                        preferred_element_type=jnp.float32)
    h = h + b_ref[...]
    h = h * jax.nn.sigmoid(h)
    mean = jnp.mean(h, axis=1, keepdims=True)
    var = jnp.mean((h - mean) ** 2, axis=1, keepdims=True)
    o_ref[...] = (h - mean) * lax.rsqrt(var + 1e-5) * g_ref[...] + bt_ref[...]


def _tc_postprocess(agg, W, b, gamma, beta):
    BM = 5000
    return pl.pallas_call(
        _tc_body,
        out_shape=jax.ShapeDtypeStruct((L * N, D), jnp.float32),
        grid=((L * N) // BM,),
        in_specs=[
            pl.BlockSpec((BM, D), lambda i: (i, 0)),
            pl.BlockSpec((D, D), lambda i: (0, 0)),
            pl.BlockSpec((1, D), lambda i: (0, 0)),
            pl.BlockSpec((1, D), lambda i: (0, 0)),
            pl.BlockSpec((1, D), lambda i: (0, 0)),
        ],
        out_specs=pl.BlockSpec((BM, D), lambda i: (i, 0)),
    )(agg, W, b.reshape(1, D), gamma.reshape(1, D), beta.reshape(1, D))


def kernel(x, edge_index, edge_weight, W, b, gamma, beta):
    x_flat = x.reshape(L * N, D)
    agg = _sc_aggregate(x_flat, edge_index.reshape(2 * E), edge_weight)
    out = _tc_postprocess(agg, W, b, gamma, beta)
    return out.reshape(L, N, D)
